# baseline (speedup 1.0000x reference)
import jax
import jax.numpy as jnp
from jax.experimental import pallas as pl


_BLK = 5000


def _body(gt_ref, pts_ref, out_ref):
    cx = gt_ref[0:1, :]
    cy = gt_ref[1:2, :]
    w = gt_ref[2:3, :]
    h = gt_ref[3:4, :]
    ang = gt_ref[4:5, :]
    cos = jnp.cos(ang)
    sin = jnp.sin(ang)
    inv_a = 2.0 / w
    inv_b = 2.0 / h
    ca = cos * inv_a
    sa = sin * inv_a
    cb = cos * inv_b
    sb = sin * inv_b
    px = pts_ref[:, 0:1]
    py = pts_ref[:, 1:2]
    dx = px - cx
    dy = py - cy
    ox = ca * dx + sa * dy
    oy = cb * dy - sb * dx
    out_ref[...] = ox * ox + oy * oy


def kernel(gt_bboxes, points):
    num_gts = gt_bboxes.shape[0]
    num_points = points.shape[0]
    gt_t = gt_bboxes.T
    grid = (num_points // _BLK,)
    return pl.pallas_call(
        _body,
        grid=grid,
        in_specs=[
            pl.BlockSpec((5, num_gts), lambda i: (0, 0)),
            pl.BlockSpec((_BLK, 2), lambda i: (i, 0)),
        ],
        out_specs=pl.BlockSpec((_BLK, num_gts), lambda i: (i, 0)),
        out_shape=jax.ShapeDtypeStruct((num_points, num_gts), jnp.float32),
    )(gt_t, points)
